# Initial kernel scaffold; baseline (speedup 1.0000x reference)
import jax
import jax.numpy as jnp
from jax.experimental import pallas as pl


def kernel(x, c1_wih0, c1_whh0, c1_bih0, c1_bhh0, c1_wih1, c1_whh1, c1_bih1, c1_bhh1, d1_wih0, d1_whh0, d1_bih0, d1_bhh0, d1_wih1, d1_whh1, d1_bih1, d1_bhh1, d2_wih0, d2_whh0, d2_bih0, d2_bhh0, d2_wih1, d2_whh1, d2_bih1, d2_bhh1, d2_wih2, d2_whh2, d2_bih2, d2_bhh2):
    raise NotImplementedError("write your pallas kernel here")


# speedup vs baseline: 1.8202x; 1.8202x over previous
import jax
import jax.numpy as jnp
from jax.experimental import pallas as pl
from jax.experimental.pallas import tpu as pltpu

S, B, E, H1, H2 = 512, 256, 300, 16, 64
BC = 128
NBC = B // BC


def _gru_update(xp, hp, h, hdim):
    r = jax.nn.sigmoid(xp[:, :hdim] + hp[:, :hdim])
    z = jax.nn.sigmoid(xp[:, hdim:2 * hdim] + hp[:, hdim:2 * hdim])
    n = jnp.tanh(xp[:, 2 * hdim:] + r * hp[:, 2 * hdim:])
    return (1.0 - z) * n + z * h


def _c1_kernel(x_ref, w1_ref, b1_ref, u1_ref, c1_ref,
               w2_ref, b2_ref, u2_ref, c2_ref,
               y_ref, h1_ref, h2_ref):
    t = pl.program_id(1)

    @pl.when(t == 0)
    def _():
        h1_ref[...] = jnp.zeros_like(h1_ref)
        h2_ref[...] = jnp.zeros_like(h2_ref)

    h1 = h1_ref[...]
    h2 = h2_ref[...]
    xt = x_ref[0]
    xp = jnp.dot(xt, w1_ref[...], preferred_element_type=jnp.float32) + b1_ref[...]
    hp = jnp.dot(h1, u1_ref[...], preferred_element_type=jnp.float32) + c1_ref[...]
    h1n = _gru_update(xp, hp, h1, H1)
    h1_ref[...] = h1n

    xp2 = jnp.dot(h1n, w2_ref[...], preferred_element_type=jnp.float32) + b2_ref[...]
    hp2 = jnp.dot(h2, u2_ref[...], preferred_element_type=jnp.float32) + c2_ref[...]
    h2n = _gru_update(xp2, hp2, h2, H1)
    h2_ref[...] = h2n
    y_ref[0] = h2n


def _dec_kernel(y_ref, h10_ref, h11_ref,
                w3_ref, b3_ref, u3_ref, c3_ref,
                w4_ref, b4_ref, u4_ref, c4_ref,
                w5_ref, b5_ref, u5_ref, c5_ref,
                w6_ref, b6_ref, u6_ref, c6_ref,
                w7_ref, b7_ref, u7_ref, c7_ref,
                out_ref,
                h3_ref, h4_ref, h5_ref, h6_ref, h7_ref):
    t = pl.program_id(1)

    @pl.when(t == 0)
    def _():
        h3_ref[...] = h10_ref[...]
        h4_ref[...] = h11_ref[...]
        h5_ref[...] = jnp.zeros_like(h5_ref)
        h6_ref[...] = jnp.zeros_like(h6_ref)
        h7_ref[...] = jnp.zeros_like(h7_ref)

    h3 = h3_ref[...]
    h4 = h4_ref[...]
    h5 = h5_ref[...]
    h6 = h6_ref[...]
    h7 = h7_ref[...]
    yt = y_ref[0]

    xp = jnp.dot(yt, w3_ref[...], preferred_element_type=jnp.float32) + b3_ref[...]
    hp = jnp.dot(h3, u3_ref[...], preferred_element_type=jnp.float32) + c3_ref[...]
    h3n = _gru_update(xp, hp, h3, H1)
    h3_ref[...] = h3n

    xp = jnp.dot(h3n, w4_ref[...], preferred_element_type=jnp.float32) + b4_ref[...]
    hp = jnp.dot(h4, u4_ref[...], preferred_element_type=jnp.float32) + c4_ref[...]
    h4n = _gru_update(xp, hp, h4, H1)
    h4_ref[...] = h4n

    xp = jnp.dot(h4n, w5_ref[...], preferred_element_type=jnp.float32) + b5_ref[...]
    hp = jnp.dot(h5, u5_ref[...], preferred_element_type=jnp.float32) + c5_ref[...]
    h5n = _gru_update(xp, hp, h5, H2)
    h5_ref[...] = h5n

    xp = jnp.dot(h5n, w6_ref[...], preferred_element_type=jnp.float32) + b6_ref[...]
    hp = jnp.dot(h6, u6_ref[...], preferred_element_type=jnp.float32) + c6_ref[...]
    h6n = _gru_update(xp, hp, h6, H2)
    h6_ref[...] = h6n

    xp = jnp.dot(h6n, w7_ref[...], preferred_element_type=jnp.float32) + b7_ref[...]
    hp = jnp.dot(h7, u7_ref[...], preferred_element_type=jnp.float32) + c7_ref[...]
    h7n = _gru_update(xp, hp, h7, H2)
    h7_ref[...] = h7n

    @pl.when(t == S - 1)
    def _():
        out_ref[:, :H2] = h7n
        out_ref[:, H2:] = h6n


def _full(shape):
    return pl.BlockSpec(shape, lambda bc, t: (0, 0))


def kernel(x,
           c1_wih0, c1_whh0, c1_bih0, c1_bhh0,
           c1_wih1, c1_whh1, c1_bih1, c1_bhh1,
           d1_wih0, d1_whh0, d1_bih0, d1_bhh0,
           d1_wih1, d1_whh1, d1_bih1, d1_bhh1,
           d2_wih0, d2_whh0, d2_bih0, d2_bhh0,
           d2_wih1, d2_whh1, d2_bih1, d2_bhh1,
           d2_wih2, d2_whh2, d2_bih2, d2_bhh2):
    f32 = jnp.float32

    def tw(w):
        return w.T

    def tb(b):
        return b.reshape(1, -1)

    y_c1, h10, h11 = pl.pallas_call(
        _c1_kernel,
        grid=(NBC, S),
        in_specs=[
            pl.BlockSpec((1, BC, E), lambda bc, t: (t, bc, 0)),
            _full((E, 3 * H1)), _full((1, 3 * H1)),
            _full((H1, 3 * H1)), _full((1, 3 * H1)),
            _full((H1, 3 * H1)), _full((1, 3 * H1)),
            _full((H1, 3 * H1)), _full((1, 3 * H1)),
        ],
        out_specs=[
            pl.BlockSpec((1, BC, H1), lambda bc, t: (t, bc, 0)),
            pl.BlockSpec((BC, H1), lambda bc, t: (bc, 0)),
            pl.BlockSpec((BC, H1), lambda bc, t: (bc, 0)),
        ],
        out_shape=[
            jax.ShapeDtypeStruct((S, B, H1), f32),
            jax.ShapeDtypeStruct((B, H1), f32),
            jax.ShapeDtypeStruct((B, H1), f32),
        ],
        compiler_params=pltpu.CompilerParams(
            dimension_semantics=("parallel", "arbitrary"),
        ),
        name="sentemb_compress1",
    )(x, tw(c1_wih0), tb(c1_bih0), tw(c1_whh0), tb(c1_bhh0),
      tw(c1_wih1), tb(c1_bih1), tw(c1_whh1), tb(c1_bhh1))

    out = pl.pallas_call(
        _dec_kernel,
        grid=(NBC, S),
        in_specs=[
            pl.BlockSpec((1, BC, H1), lambda bc, t: (t, bc, 0)),
            pl.BlockSpec((BC, H1), lambda bc, t: (bc, 0)),
            pl.BlockSpec((BC, H1), lambda bc, t: (bc, 0)),
            _full((H1, 3 * H1)), _full((1, 3 * H1)),
            _full((H1, 3 * H1)), _full((1, 3 * H1)),
            _full((H1, 3 * H1)), _full((1, 3 * H1)),
            _full((H1, 3 * H1)), _full((1, 3 * H1)),
            _full((H1, 3 * H2)), _full((1, 3 * H2)),
            _full((H2, 3 * H2)), _full((1, 3 * H2)),
            _full((H2, 3 * H2)), _full((1, 3 * H2)),
            _full((H2, 3 * H2)), _full((1, 3 * H2)),
            _full((H2, 3 * H2)), _full((1, 3 * H2)),
            _full((H2, 3 * H2)), _full((1, 3 * H2)),
        ],
        out_specs=pl.BlockSpec((BC, 2 * H2), lambda bc, t: (bc, 0)),
        out_shape=jax.ShapeDtypeStruct((B, 2 * H2), f32),
        scratch_shapes=[
            pltpu.VMEM((BC, H1), f32),
            pltpu.VMEM((BC, H1), f32),
            pltpu.VMEM((BC, H2), f32),
            pltpu.VMEM((BC, H2), f32),
            pltpu.VMEM((BC, H2), f32),
        ],
        compiler_params=pltpu.CompilerParams(
            dimension_semantics=("parallel", "arbitrary"),
        ),
        name="sentemb_decode",
    )(y_c1, h10, h11,
      tw(d1_wih0), tb(d1_bih0), tw(d1_whh0), tb(d1_bhh0),
      tw(d1_wih1), tb(d1_bih1), tw(d1_whh1), tb(d1_bhh1),
      tw(d2_wih0), tb(d2_bih0), tw(d2_whh0), tb(d2_bhh0),
      tw(d2_wih1), tb(d2_bih1), tw(d2_whh1), tb(d2_bhh1),
      tw(d2_wih2), tb(d2_bih2), tw(d2_whh2), tb(d2_bhh2))

    return out


# speedup vs baseline: 6.1582x; 3.3833x over previous
import jax
import jax.numpy as jnp
from jax.experimental import pallas as pl
from jax.experimental.pallas import tpu as pltpu

S, B, E, H1, H2 = 512, 256, 300, 16, 64


def _gru_t(xp, hp, h, hdim):
    r = jax.nn.sigmoid(xp[:hdim] + hp[:hdim])
    z = jax.nn.sigmoid(xp[hdim:2 * hdim] + hp[hdim:2 * hdim])
    n = jnp.tanh(xp[2 * hdim:] + r * hp[2 * hdim:])
    return (1.0 - z) * n + z * h


def _dot(a, b):
    return jnp.dot(a, b, preferred_element_type=jnp.float32)


def _c1_kernel(x_ref, w1_ref, b1_ref, u1_ref, c1_ref,
               w2_ref, b2_ref, u2_ref, c2_ref,
               y_ref, h1_ref, h2_ref):
    i = pl.program_id(0)

    @pl.when(i == 0)
    def _():
        h1_ref[...] = jnp.zeros_like(h1_ref)
        h2_ref[...] = jnp.zeros_like(h2_ref)

    h1 = h1_ref[...]
    h2 = h2_ref[...]

    xp1 = jax.lax.dot_general(
        w1_ref[...], x_ref[0], (((1,), (1,)), ((), ())),
        preferred_element_type=jnp.float32) + b1_ref[...]
    hp1 = _dot(u1_ref[...], h1) + c1_ref[...]
    h1n = _gru_t(xp1, hp1, h1, H1)

    xp2 = _dot(w2_ref[...], h1) + b2_ref[...]
    hp2 = _dot(u2_ref[...], h2) + c2_ref[...]
    h2n = _gru_t(xp2, hp2, h2, H1)

    h1_ref[...] = jnp.where(i < S, h1n, h1)
    h2_ref[...] = jnp.where(i >= 1, h2n, h2)
    y_ref[0] = h2n


def _dec_kernel(y_ref, h10_ref, h11_ref,
                w3_ref, b3_ref, u3_ref, c3_ref,
                w4_ref, b4_ref, u4_ref, c4_ref,
                w5_ref, b5_ref, u5_ref, c5_ref,
                w6_ref, b6_ref, u6_ref, c6_ref,
                w7_ref, b7_ref, u7_ref, c7_ref,
                out_ref,
                h3_ref, h4_ref, h5_ref, h6_ref, h7_ref):
    i = pl.program_id(0)

    @pl.when(i == 0)
    def _():
        h3_ref[...] = h10_ref[...]
        h4_ref[...] = h11_ref[...]
        h5_ref[...] = jnp.zeros_like(h5_ref)
        h6_ref[...] = jnp.zeros_like(h6_ref)
        h7_ref[...] = jnp.zeros_like(h7_ref)

    h3 = h3_ref[...]
    h4 = h4_ref[...]
    h5 = h5_ref[...]
    h6 = h6_ref[...]
    h7 = h7_ref[...]

    xp = _dot(w3_ref[...], y_ref[0]) + b3_ref[...]
    hp = _dot(u3_ref[...], h3) + c3_ref[...]
    h3n = _gru_t(xp, hp, h3, H1)

    xp = _dot(w4_ref[...], h3) + b4_ref[...]
    hp = _dot(u4_ref[...], h4) + c4_ref[...]
    h4n = _gru_t(xp, hp, h4, H1)

    xp = _dot(w5_ref[...], h4) + b5_ref[...]
    hp = _dot(u5_ref[...], h5) + c5_ref[...]
    h5n = _gru_t(xp, hp, h5, H2)

    xp = _dot(w6_ref[...], h5) + b6_ref[...]
    hp = _dot(u6_ref[...], h6) + c6_ref[...]
    h6n = _gru_t(xp, hp, h6, H2)

    xp = _dot(w7_ref[...], h6) + b7_ref[...]
    hp = _dot(u7_ref[...], h7) + c7_ref[...]
    h7n = _gru_t(xp, hp, h7, H2)

    h3_ref[...] = jnp.where(i < S, h3n, h3)
    h4_ref[...] = jnp.where(jnp.logical_and(i >= 1, i < S + 1), h4n, h4)
    h5_ref[...] = jnp.where(jnp.logical_and(i >= 2, i < S + 2), h5n, h5)
    h6_ref[...] = jnp.where(jnp.logical_and(i >= 3, i < S + 3), h6n, h6)
    h7_ref[...] = jnp.where(i >= 4, h7n, h7)

    @pl.when(i == S + 3)
    def _():
        out_ref[:H2, :] = h7n
        out_ref[H2:, :] = h6_ref[...]


def _full2(shape):
    return pl.BlockSpec(shape, lambda i: (0, 0))


def kernel(x,
           c1_wih0, c1_whh0, c1_bih0, c1_bhh0,
           c1_wih1, c1_whh1, c1_bih1, c1_bhh1,
           d1_wih0, d1_whh0, d1_bih0, d1_bhh0,
           d1_wih1, d1_whh1, d1_bih1, d1_bhh1,
           d2_wih0, d2_whh0, d2_bih0, d2_bhh0,
           d2_wih1, d2_whh1, d2_bih1, d2_bhh1,
           d2_wih2, d2_whh2, d2_bih2, d2_bhh2):
    f32 = jnp.float32

    def bb(b):
        return jnp.broadcast_to(b.reshape(-1, 1), (b.shape[0], B))

    y_c1, h10, h11 = pl.pallas_call(
        _c1_kernel,
        grid=(S + 1,),
        in_specs=[
            pl.BlockSpec((1, B, E), lambda i: (jnp.minimum(i, S - 1), 0, 0)),
            _full2((3 * H1, E)), _full2((3 * H1, B)),
            _full2((3 * H1, H1)), _full2((3 * H1, B)),
            _full2((3 * H1, H1)), _full2((3 * H1, B)),
            _full2((3 * H1, H1)), _full2((3 * H1, B)),
        ],
        out_specs=[
            pl.BlockSpec((1, H1, B), lambda i: (jnp.maximum(i - 1, 0), 0, 0)),
            pl.BlockSpec((H1, B), lambda i: (0, 0)),
            pl.BlockSpec((H1, B), lambda i: (0, 0)),
        ],
        out_shape=[
            jax.ShapeDtypeStruct((S, H1, B), f32),
            jax.ShapeDtypeStruct((H1, B), f32),
            jax.ShapeDtypeStruct((H1, B), f32),
        ],
        compiler_params=pltpu.CompilerParams(
            dimension_semantics=("arbitrary",),
        ),
        name="sentemb_compress1",
    )(x, c1_wih0, bb(c1_bih0), c1_whh0, bb(c1_bhh0),
      c1_wih1, bb(c1_bih1), c1_whh1, bb(c1_bhh1))

    outT = pl.pallas_call(
        _dec_kernel,
        grid=(S + 4,),
        in_specs=[
            pl.BlockSpec((1, H1, B), lambda i: (jnp.minimum(i, S - 1), 0, 0)),
            pl.BlockSpec((H1, B), lambda i: (0, 0)),
            pl.BlockSpec((H1, B), lambda i: (0, 0)),
            _full2((3 * H1, H1)), _full2((3 * H1, B)),
            _full2((3 * H1, H1)), _full2((3 * H1, B)),
            _full2((3 * H1, H1)), _full2((3 * H1, B)),
            _full2((3 * H1, H1)), _full2((3 * H1, B)),
            _full2((3 * H2, H1)), _full2((3 * H2, B)),
            _full2((3 * H2, H2)), _full2((3 * H2, B)),
            _full2((3 * H2, H2)), _full2((3 * H2, B)),
            _full2((3 * H2, H2)), _full2((3 * H2, B)),
            _full2((3 * H2, H2)), _full2((3 * H2, B)),
            _full2((3 * H2, H2)), _full2((3 * H2, B)),
        ],
        out_specs=pl.BlockSpec((2 * H2, B), lambda i: (0, 0)),
        out_shape=jax.ShapeDtypeStruct((2 * H2, B), f32),
        scratch_shapes=[
            pltpu.VMEM((H1, B), f32),
            pltpu.VMEM((H1, B), f32),
            pltpu.VMEM((H2, B), f32),
            pltpu.VMEM((H2, B), f32),
            pltpu.VMEM((H2, B), f32),
        ],
        compiler_params=pltpu.CompilerParams(
            dimension_semantics=("arbitrary",),
        ),
        name="sentemb_decode",
    )(y_c1, h10, h11,
      d1_wih0, bb(d1_bih0), d1_whh0, bb(d1_bhh0),
      d1_wih1, bb(d1_bih1), d1_whh1, bb(d1_bhh1),
      d2_wih0, bb(d2_bih0), d2_whh0, bb(d2_bhh0),
      d2_wih1, bb(d2_bih1), d2_whh1, bb(d2_bhh1),
      d2_wih2, bb(d2_bih2), d2_whh2, bb(d2_bhh2))

    return outT.T


# speedup vs baseline: 12.0625x; 1.9588x over previous
import jax
import jax.numpy as jnp
from jax.experimental import pallas as pl
from jax.experimental.pallas import tpu as pltpu

S, B, E, H1, H2 = 512, 256, 300, 16, 64
TBLK = 8
NBLK = S // TBLK + 1
NI = NBLK * TBLK


def _gru_t(xp, hp, h, hdim):
    r = jax.nn.sigmoid(xp[:hdim] + hp[:hdim])
    z = jax.nn.sigmoid(xp[hdim:2 * hdim] + hp[hdim:2 * hdim])
    n = jnp.tanh(xp[2 * hdim:] + r * hp[2 * hdim:])
    return (1.0 - z) * n + z * h


def _dot(a, b):
    return jnp.dot(a, b, preferred_element_type=jnp.float32)


def _c1_kernel(x_ref, w1_ref, b1_ref, u1_ref, c1_ref,
               w2_ref, b2_ref, u2_ref, c2_ref,
               y_ref, h1_ref, h2_ref):
    blk = pl.program_id(0)

    @pl.when(blk == 0)
    def _():
        h1_ref[...] = jnp.zeros_like(h1_ref)
        h2_ref[...] = jnp.zeros_like(h2_ref)

    h1 = h1_ref[...]
    h2 = h2_ref[...]
    for dt in range(TBLK):
        i = blk * TBLK + dt
        xp1 = jax.lax.dot_general(
            w1_ref[...], x_ref[dt], (((1,), (1,)), ((), ())),
            preferred_element_type=jnp.float32) + b1_ref[...]
        hp1 = _dot(u1_ref[...], h1) + c1_ref[...]
        h1n = _gru_t(xp1, hp1, h1, H1)
        xp2 = _dot(w2_ref[...], h1) + b2_ref[...]
        hp2 = _dot(u2_ref[...], h2) + c2_ref[...]
        h2n = _gru_t(xp2, hp2, h2, H1)
        h1 = jnp.where(i < S, h1n, h1)
        h2 = jnp.where(jnp.logical_and(i >= 1, i < S + 1), h2n, h2)
        y_ref[dt] = h2n
    h1_ref[...] = h1
    h2_ref[...] = h2


def _dec_kernel(y_ref, h10_ref, h11_ref,
                w3_ref, b3_ref, u3_ref, c3_ref,
                w4_ref, b4_ref, u4_ref, c4_ref,
                w5_ref, b5_ref, u5_ref, c5_ref,
                w6_ref, b6_ref, u6_ref, c6_ref,
                w7_ref, b7_ref, u7_ref, c7_ref,
                out_ref,
                h3_ref, h4_ref, h5_ref, h6_ref, h7_ref):
    blk = pl.program_id(0)

    @pl.when(blk == 0)
    def _():
        h3_ref[...] = h10_ref[...]
        h4_ref[...] = h11_ref[...]
        h5_ref[...] = jnp.zeros_like(h5_ref)
        h6_ref[...] = jnp.zeros_like(h6_ref)
        h7_ref[...] = jnp.zeros_like(h7_ref)

    h3 = h3_ref[...]
    h4 = h4_ref[...]
    h5 = h5_ref[...]
    h6 = h6_ref[...]
    h7 = h7_ref[...]
    for dt in range(TBLK):
        j = blk * TBLK + dt
        xp = _dot(w3_ref[...], y_ref[dt]) + b3_ref[...]
        hp = _dot(u3_ref[...], h3) + c3_ref[...]
        h3n = _gru_t(xp, hp, h3, H1)

        xp = _dot(w4_ref[...], h3) + b4_ref[...]
        hp = _dot(u4_ref[...], h4) + c4_ref[...]
        h4n = _gru_t(xp, hp, h4, H1)

        xp = _dot(w5_ref[...], h4) + b5_ref[...]
        hp = _dot(u5_ref[...], h5) + c5_ref[...]
        h5n = _gru_t(xp, hp, h5, H2)

        xp = _dot(w6_ref[...], h5) + b6_ref[...]
        hp = _dot(u6_ref[...], h6) + c6_ref[...]
        h6n = _gru_t(xp, hp, h6, H2)

        xp = _dot(w7_ref[...], h6) + b7_ref[...]
        hp = _dot(u7_ref[...], h7) + c7_ref[...]
        h7n = _gru_t(xp, hp, h7, H2)

        h3 = jnp.where(jnp.logical_and(j >= 1, j < S + 1), h3n, h3)
        h4 = jnp.where(jnp.logical_and(j >= 2, j < S + 2), h4n, h4)
        h5 = jnp.where(jnp.logical_and(j >= 3, j < S + 3), h5n, h5)
        h6 = jnp.where(jnp.logical_and(j >= 4, j < S + 4), h6n, h6)
        h7 = jnp.where(jnp.logical_and(j >= 5, j < S + 5), h7n, h7)
    h3_ref[...] = h3
    h4_ref[...] = h4
    h5_ref[...] = h5
    h6_ref[...] = h6
    h7_ref[...] = h7

    @pl.when(blk == NBLK - 1)
    def _():
        out_ref[:H2, :] = h7
        out_ref[H2:, :] = h6


def _full2(shape):
    return pl.BlockSpec(shape, lambda i: (0, 0))


def kernel(x,
           c1_wih0, c1_whh0, c1_bih0, c1_bhh0,
           c1_wih1, c1_whh1, c1_bih1, c1_bhh1,
           d1_wih0, d1_whh0, d1_bih0, d1_bhh0,
           d1_wih1, d1_whh1, d1_bih1, d1_bhh1,
           d2_wih0, d2_whh0, d2_bih0, d2_bhh0,
           d2_wih1, d2_whh1, d2_bih1, d2_bhh1,
           d2_wih2, d2_whh2, d2_bih2, d2_bhh2):
    f32 = jnp.float32

    def bb(b):
        return jnp.broadcast_to(b.reshape(-1, 1), (b.shape[0], B))

    y_pad, h10, h11 = pl.pallas_call(
        _c1_kernel,
        grid=(NBLK,),
        in_specs=[
            pl.BlockSpec((TBLK, B, E), lambda i: (jnp.minimum(i, S // TBLK - 1), 0, 0)),
            _full2((3 * H1, E)), _full2((3 * H1, B)),
            _full2((3 * H1, H1)), _full2((3 * H1, B)),
            _full2((3 * H1, H1)), _full2((3 * H1, B)),
            _full2((3 * H1, H1)), _full2((3 * H1, B)),
        ],
        out_specs=[
            pl.BlockSpec((TBLK, H1, B), lambda i: (i, 0, 0)),
            pl.BlockSpec((H1, B), lambda i: (0, 0)),
            pl.BlockSpec((H1, B), lambda i: (0, 0)),
        ],
        out_shape=[
            jax.ShapeDtypeStruct((NI, H1, B), f32),
            jax.ShapeDtypeStruct((H1, B), f32),
            jax.ShapeDtypeStruct((H1, B), f32),
        ],
        compiler_params=pltpu.CompilerParams(
            dimension_semantics=("arbitrary",),
        ),
        name="sentemb_compress1",
    )(x, c1_wih0, bb(c1_bih0), c1_whh0, bb(c1_bhh0),
      c1_wih1, bb(c1_bih1), c1_whh1, bb(c1_bhh1))

    outT = pl.pallas_call(
        _dec_kernel,
        grid=(NBLK,),
        in_specs=[
            pl.BlockSpec((TBLK, H1, B), lambda i: (i, 0, 0)),
            pl.BlockSpec((H1, B), lambda i: (0, 0)),
            pl.BlockSpec((H1, B), lambda i: (0, 0)),
            _full2((3 * H1, H1)), _full2((3 * H1, B)),
            _full2((3 * H1, H1)), _full2((3 * H1, B)),
            _full2((3 * H1, H1)), _full2((3 * H1, B)),
            _full2((3 * H1, H1)), _full2((3 * H1, B)),
            _full2((3 * H2, H1)), _full2((3 * H2, B)),
            _full2((3 * H2, H2)), _full2((3 * H2, B)),
            _full2((3 * H2, H2)), _full2((3 * H2, B)),
            _full2((3 * H2, H2)), _full2((3 * H2, B)),
            _full2((3 * H2, H2)), _full2((3 * H2, B)),
            _full2((3 * H2, H2)), _full2((3 * H2, B)),
        ],
        out_specs=pl.BlockSpec((2 * H2, B), lambda i: (0, 0)),
        out_shape=jax.ShapeDtypeStruct((2 * H2, B), f32),
        scratch_shapes=[
            pltpu.VMEM((H1, B), f32),
            pltpu.VMEM((H1, B), f32),
            pltpu.VMEM((H2, B), f32),
            pltpu.VMEM((H2, B), f32),
            pltpu.VMEM((H2, B), f32),
        ],
        compiler_params=pltpu.CompilerParams(
            dimension_semantics=("arbitrary",),
        ),
        name="sentemb_decode",
    )(y_pad, h10, h11,
      d1_wih0, bb(d1_bih0), d1_whh0, bb(d1_bhh0),
      d1_wih1, bb(d1_bih1), d1_whh1, bb(d1_bhh1),
      d2_wih0, bb(d2_bih0), d2_whh0, bb(d2_bhh0),
      d2_wih1, bb(d2_bih1), d2_whh1, bb(d2_bhh1),
      d2_wih2, bb(d2_bih2), d2_whh2, bb(d2_bhh2))

    return outT.T


# speedup vs baseline: 12.7426x; 1.0564x over previous
import jax
import jax.numpy as jnp
from jax.experimental import pallas as pl
from jax.experimental.pallas import tpu as pltpu

S, B, E, H1, H2 = 512, 256, 300, 16, 64
TBLK = 16
NBLK = S // TBLK + 1
NI = NBLK * TBLK


def _gru_t(xp, hp, h, hdim):
    r = jax.nn.sigmoid(xp[:hdim] + hp[:hdim])
    z = jax.nn.sigmoid(xp[hdim:2 * hdim] + hp[hdim:2 * hdim])
    n = jnp.tanh(xp[2 * hdim:] + r * hp[2 * hdim:])
    return (1.0 - z) * n + z * h


def _dot(a, b):
    return jnp.dot(a, b, preferred_element_type=jnp.float32)


def _c1_kernel(x_ref, w1_ref, b1_ref, u1_ref, c1_ref,
               w2_ref, b2_ref, u2_ref, c2_ref,
               y_ref, h1_ref, h2_ref):
    blk = pl.program_id(0)

    @pl.when(blk == 0)
    def _():
        h1_ref[...] = jnp.zeros_like(h1_ref)
        h2_ref[...] = jnp.zeros_like(h2_ref)

    h1 = h1_ref[...]
    h2 = h2_ref[...]
    for dt in range(TBLK):
        i = blk * TBLK + dt
        xp1 = jax.lax.dot_general(
            w1_ref[...], x_ref[dt], (((1,), (1,)), ((), ())),
            preferred_element_type=jnp.float32) + b1_ref[...]
        hp1 = _dot(u1_ref[...], h1) + c1_ref[...]
        h1n = _gru_t(xp1, hp1, h1, H1)
        xp2 = _dot(w2_ref[...], h1) + b2_ref[...]
        hp2 = _dot(u2_ref[...], h2) + c2_ref[...]
        h2n = _gru_t(xp2, hp2, h2, H1)
        h1 = jnp.where(i < S, h1n, h1)
        h2 = jnp.where(jnp.logical_and(i >= 1, i < S + 1), h2n, h2)
        y_ref[dt] = h2n
    h1_ref[...] = h1
    h2_ref[...] = h2


def _dec_kernel(y_ref, h10_ref, h11_ref,
                w3_ref, b3_ref, u3_ref, c3_ref,
                w4_ref, b4_ref, u4_ref, c4_ref,
                w5_ref, b5_ref, u5_ref, c5_ref,
                w6_ref, b6_ref, u6_ref, c6_ref,
                w7_ref, b7_ref, u7_ref, c7_ref,
                out_ref,
                h3_ref, h4_ref, h5_ref, h6_ref, h7_ref):
    blk = pl.program_id(0)

    @pl.when(blk == 0)
    def _():
        h3_ref[...] = h10_ref[...]
        h4_ref[...] = h11_ref[...]
        h5_ref[...] = jnp.zeros_like(h5_ref)
        h6_ref[...] = jnp.zeros_like(h6_ref)
        h7_ref[...] = jnp.zeros_like(h7_ref)

    h3 = h3_ref[...]
    h4 = h4_ref[...]
    h5 = h5_ref[...]
    h6 = h6_ref[...]
    h7 = h7_ref[...]
    for dt in range(TBLK):
        j = blk * TBLK + dt
        xp = _dot(w3_ref[...], y_ref[dt]) + b3_ref[...]
        hp = _dot(u3_ref[...], h3) + c3_ref[...]
        h3n = _gru_t(xp, hp, h3, H1)

        xp = _dot(w4_ref[...], h3) + b4_ref[...]
        hp = _dot(u4_ref[...], h4) + c4_ref[...]
        h4n = _gru_t(xp, hp, h4, H1)

        xp = _dot(w5_ref[...], h4) + b5_ref[...]
        hp = _dot(u5_ref[...], h5) + c5_ref[...]
        h5n = _gru_t(xp, hp, h5, H2)

        xp = _dot(w6_ref[...], h5) + b6_ref[...]
        hp = _dot(u6_ref[...], h6) + c6_ref[...]
        h6n = _gru_t(xp, hp, h6, H2)

        xp = _dot(w7_ref[...], h6) + b7_ref[...]
        hp = _dot(u7_ref[...], h7) + c7_ref[...]
        h7n = _gru_t(xp, hp, h7, H2)

        h3 = jnp.where(jnp.logical_and(j >= 1, j < S + 1), h3n, h3)
        h4 = jnp.where(jnp.logical_and(j >= 2, j < S + 2), h4n, h4)
        h5 = jnp.where(jnp.logical_and(j >= 3, j < S + 3), h5n, h5)
        h6 = jnp.where(jnp.logical_and(j >= 4, j < S + 4), h6n, h6)
        h7 = jnp.where(jnp.logical_and(j >= 5, j < S + 5), h7n, h7)
    h3_ref[...] = h3
    h4_ref[...] = h4
    h5_ref[...] = h5
    h6_ref[...] = h6
    h7_ref[...] = h7

    @pl.when(blk == NBLK - 1)
    def _():
        out_ref[:H2, :] = h7
        out_ref[H2:, :] = h6


def _full2(shape):
    return pl.BlockSpec(shape, lambda i: (0, 0))


def kernel(x,
           c1_wih0, c1_whh0, c1_bih0, c1_bhh0,
           c1_wih1, c1_whh1, c1_bih1, c1_bhh1,
           d1_wih0, d1_whh0, d1_bih0, d1_bhh0,
           d1_wih1, d1_whh1, d1_bih1, d1_bhh1,
           d2_wih0, d2_whh0, d2_bih0, d2_bhh0,
           d2_wih1, d2_whh1, d2_bih1, d2_bhh1,
           d2_wih2, d2_whh2, d2_bih2, d2_bhh2):
    f32 = jnp.float32

    def bb(b):
        return jnp.broadcast_to(b.reshape(-1, 1), (b.shape[0], B))

    y_pad, h10, h11 = pl.pallas_call(
        _c1_kernel,
        grid=(NBLK,),
        in_specs=[
            pl.BlockSpec((TBLK, B, E), lambda i: (jnp.minimum(i, S // TBLK - 1), 0, 0)),
            _full2((3 * H1, E)), _full2((3 * H1, B)),
            _full2((3 * H1, H1)), _full2((3 * H1, B)),
            _full2((3 * H1, H1)), _full2((3 * H1, B)),
            _full2((3 * H1, H1)), _full2((3 * H1, B)),
        ],
        out_specs=[
            pl.BlockSpec((TBLK, H1, B), lambda i: (i, 0, 0)),
            pl.BlockSpec((H1, B), lambda i: (0, 0)),
            pl.BlockSpec((H1, B), lambda i: (0, 0)),
        ],
        out_shape=[
            jax.ShapeDtypeStruct((NI, H1, B), f32),
            jax.ShapeDtypeStruct((H1, B), f32),
            jax.ShapeDtypeStruct((H1, B), f32),
        ],
        compiler_params=pltpu.CompilerParams(
            dimension_semantics=("arbitrary",),
        ),
        name="sentemb_compress1",
    )(x, c1_wih0, bb(c1_bih0), c1_whh0, bb(c1_bhh0),
      c1_wih1, bb(c1_bih1), c1_whh1, bb(c1_bhh1))

    outT = pl.pallas_call(
        _dec_kernel,
        grid=(NBLK,),
        in_specs=[
            pl.BlockSpec((TBLK, H1, B), lambda i: (i, 0, 0)),
            pl.BlockSpec((H1, B), lambda i: (0, 0)),
            pl.BlockSpec((H1, B), lambda i: (0, 0)),
            _full2((3 * H1, H1)), _full2((3 * H1, B)),
            _full2((3 * H1, H1)), _full2((3 * H1, B)),
            _full2((3 * H1, H1)), _full2((3 * H1, B)),
            _full2((3 * H1, H1)), _full2((3 * H1, B)),
            _full2((3 * H2, H1)), _full2((3 * H2, B)),
            _full2((3 * H2, H2)), _full2((3 * H2, B)),
            _full2((3 * H2, H2)), _full2((3 * H2, B)),
            _full2((3 * H2, H2)), _full2((3 * H2, B)),
            _full2((3 * H2, H2)), _full2((3 * H2, B)),
            _full2((3 * H2, H2)), _full2((3 * H2, B)),
        ],
        out_specs=pl.BlockSpec((2 * H2, B), lambda i: (0, 0)),
        out_shape=jax.ShapeDtypeStruct((2 * H2, B), f32),
        scratch_shapes=[
            pltpu.VMEM((H1, B), f32),
            pltpu.VMEM((H1, B), f32),
            pltpu.VMEM((H2, B), f32),
            pltpu.VMEM((H2, B), f32),
            pltpu.VMEM((H2, B), f32),
        ],
        compiler_params=pltpu.CompilerParams(
            dimension_semantics=("arbitrary",),
        ),
        name="sentemb_decode",
    )(y_pad, h10, h11,
      d1_wih0, bb(d1_bih0), d1_whh0, bb(d1_bhh0),
      d1_wih1, bb(d1_bih1), d1_whh1, bb(d1_bhh1),
      d2_wih0, bb(d2_bih0), d2_whh0, bb(d2_bhh0),
      d2_wih1, bb(d2_bih1), d2_whh1, bb(d2_bhh1),
      d2_wih2, bb(d2_bih2), d2_whh2, bb(d2_bhh2))

    return outT.T
